# baseline (speedup 1.0000x reference)
import functools

import jax
import jax.numpy as jnp
from jax.experimental import pallas as pl
from jax.experimental.pallas import tpu as pltpu

EPS = 1e-6
BETA_CE = 1.0
BETA_CU = 1.0


def _mm_kernel(x_ref, w_ref, o_ref, acc_ref, *, nk):
    @pl.when(pl.program_id(2) == 0)
    def _init():
        acc_ref[...] = jnp.zeros_like(acc_ref)

    acc_ref[...] += jnp.dot(x_ref[...], w_ref[...],
                            preferred_element_type=jnp.float32)

    @pl.when(pl.program_id(2) == nk - 1)
    def _done():
        o_ref[...] = acc_ref[...].astype(o_ref.dtype)


def _mm(x, w, bm=1024, bn=2048, bk=512, out_dtype=None):
    m, k = x.shape
    _, n = w.shape
    bm, bn, bk = min(bm, m), min(bn, n), min(bk, k)
    nm, nn, nk = m // bm, n // bn, k // bk
    out_dtype = out_dtype or x.dtype
    return pl.pallas_call(
        functools.partial(_mm_kernel, nk=nk),
        grid=(nm, nn, nk),
        in_specs=[
            pl.BlockSpec((bm, bk), lambda i, j, kk: (i, kk)),
            pl.BlockSpec((bk, bn), lambda i, j, kk: (kk, j)),
        ],
        out_specs=pl.BlockSpec((bm, bn), lambda i, j, kk: (i, j)),
        out_shape=jax.ShapeDtypeStruct((m, n), out_dtype),
        scratch_shapes=[pltpu.VMEM((bm, bn), jnp.float32)],
        compiler_params=pltpu.CompilerParams(
            dimension_semantics=("parallel", "parallel", "arbitrary")),
    )(x, w)


def _glu_kernel(x_ref, wg_ref, wu_ref, h_ref):
    x = x_ref[...]
    g = jnp.dot(x, wg_ref[...], preferred_element_type=jnp.float32)
    u = jnp.dot(x, wu_ref[...], preferred_element_type=jnp.float32)
    h_ref[...] = (g * jax.nn.sigmoid(g) * u).astype(h_ref.dtype)


def _glu(x, wg, wu, bm=1024, bn=512):
    m, k = x.shape
    _, n = wg.shape
    bm, bn = min(bm, m), min(bn, n)
    return pl.pallas_call(
        _glu_kernel,
        grid=(m // bm, n // bn),
        in_specs=[
            pl.BlockSpec((bm, k), lambda i, j: (i, 0)),
            pl.BlockSpec((k, bn), lambda i, j: (0, j)),
            pl.BlockSpec((k, bn), lambda i, j: (0, j)),
        ],
        out_specs=pl.BlockSpec((bm, bn), lambda i, j: (i, j)),
        out_shape=jax.ShapeDtypeStruct((m, n), x.dtype),
        compiler_params=pltpu.CompilerParams(
            dimension_semantics=("parallel", "parallel")),
    )(x, wg, wu)


def _attn_blk_kernel(q_ref, k_ref, v_ref, o_ref, *, qi, bq, scale):
    s = jax.lax.dot_general(q_ref[...], k_ref[...], (((1,), (1,)), ((), ())),
                            preferred_element_type=jnp.float32) * scale
    pref = k_ref.shape[0]
    rows = qi * bq + jax.lax.broadcasted_iota(jnp.int32, (bq, pref), 0)
    cols = jax.lax.broadcasted_iota(jnp.int32, (bq, pref), 1)
    s = jnp.where(cols <= rows, s, -1e30)
    m = jnp.max(s, axis=1, keepdims=True)
    p = jnp.exp(s - m)
    a = p / jnp.sum(p, axis=1, keepdims=True)
    o_ref[...] = jnp.dot(a, v_ref[...], preferred_element_type=jnp.float32)


def _attention(q, k, v, h, hd, bq=1024):
    t, d = q.shape
    bq = min(bq, t)
    scale = 1.0 / (hd ** 0.5)
    outs = []
    for qi in range(t // bq):
        pref = (qi + 1) * bq
        o = pl.pallas_call(
            functools.partial(_attn_blk_kernel, qi=qi, bq=bq, scale=scale),
            grid=(h,),
            in_specs=[
                pl.BlockSpec((bq, hd), lambda hh, _qi=qi: (_qi, hh)),
                pl.BlockSpec((pref, hd), lambda hh: (0, hh)),
                pl.BlockSpec((pref, hd), lambda hh: (0, hh)),
            ],
            out_specs=pl.BlockSpec((bq, hd), lambda hh: (0, hh)),
            out_shape=jax.ShapeDtypeStruct((bq, d), jnp.float32),
            compiler_params=pltpu.CompilerParams(
                dimension_semantics=("arbitrary",)),
        )(q, k, v)
        outs.append(o)
    return outs[0] if len(outs) == 1 else jnp.concatenate(outs, axis=0)


def _rms(x, w):
    return x * jax.lax.rsqrt(jnp.mean(x * x, axis=-1, keepdims=True) + EPS) * w


def _rope_cos_sin(t, hd):
    inv = 1.0 / (10000.0 ** (jnp.arange(0, hd, 2, dtype=jnp.float32) / hd))
    freqs = jnp.arange(t, dtype=jnp.float32)[:, None] * inv[None, :]
    cos = jnp.concatenate([jnp.cos(freqs), jnp.cos(freqs)], axis=-1)
    sin = jnp.concatenate([jnp.sin(freqs), jnp.sin(freqs)], axis=-1)
    return cos, sin


def _rope_tD(x, cos, sin, h, hd):
    t, d = x.shape
    xh = x.reshape(t, h, hd)
    x1 = xh[..., :hd // 2]
    x2 = xh[..., hd // 2:]
    rot = jnp.concatenate([-x2, x1], axis=-1)
    out = xh * cos[:, None, :] + rot * sin[:, None, :]
    return out.reshape(t, d)


def _decoder(x, p, pref, h, hd, cos, sin):
    xn = _rms(x, p[pref + 'ln1'])
    q = _mm(xn, p[pref + 'wq'])
    k = _mm(xn, p[pref + 'wk'])
    v = _mm(xn, p[pref + 'wv'])
    q = _rope_tD(q, cos, sin, h, hd)
    k = _rope_tD(k, cos, sin, h, hd)
    ao = _attention(q, k, v, h, hd)
    x = x + _mm(ao, p[pref + 'wo'])
    hn = _rms(x, p[pref + 'ln2'])
    hh = _glu(hn, p[pref + 'wg'], p[pref + 'wu'])
    return x + _mm(hh, p[pref + 'wd'])


def kernel(hidden_states, params):
    p = params
    b, t, d = hidden_states.shape
    x = hidden_states.reshape(t, d)
    h = 16
    hd = d // h
    cos, sin = _rope_cos_sin(t, hd)

    processed = _decoder(x, p, 'l1_', h, hd, cos, sin)
    pn = _rms(x, p['p_ln'])
    ph = _glu(pn, p['p_wg'], p['p_wu'])
    prior_out = _mm(ph, p['p_wd'])
    prior_hidden = x + prior_out

    prior_loss = jnp.mean((prior_hidden - processed) ** 2)

    actual = processed - x
    predicted = prior_out
    D_st = jnp.sum(actual ** 2, axis=-1) / d
    D_ch = jnp.sum((actual - predicted) ** 2, axis=-1) / d
    z_st = (D_st - jnp.mean(D_st)) / (jnp.std(D_st) + 1e-6)
    z_ch = (D_ch - jnp.mean(D_ch)) / (jnp.std(D_ch) + 1e-6)
    g_cont = jax.nn.sigmoid(BETA_CE * z_st - BETA_CU * z_ch)

    kk = max(1, int(t * 0.5))
    gscores, topk_idx = jax.lax.top_k(g_cont, kk)

    binary = jnp.zeros((t,), jnp.float32).at[topk_idx].set(1.0)
    logits = x @ p['r_w']
    causal_loss = jnp.mean(jnp.maximum(logits, 0.0) - logits * binary
                           + jnp.log1p(jnp.exp(-jnp.abs(logits))))

    sel = processed[topk_idx]
    out2 = _decoder(sel, p, 'l2_', h, hd, cos[:kk], sin[:kk])
    new = sel + gscores[:, None] * (out2 - sel)
    final = processed.at[topk_idx].set(new)

    return final.reshape(b, t, d), prior_loss, causal_loss


# speedup vs baseline: 1.0233x; 1.0233x over previous
import functools

import jax
import jax.numpy as jnp
from jax import lax
from jax.experimental import pallas as pl
from jax.experimental.pallas import tpu as pltpu
from jax.experimental.pallas import tpu_sc as plsc

EPS = 1e-6
BETA_CE = 1.0
BETA_CU = 1.0


def _split_bf16(x):
    hi = x.astype(jnp.bfloat16)
    lo = (x - hi.astype(jnp.float32)).astype(jnp.bfloat16)
    return hi, lo


def _dot3(x, w):
    return jnp.dot(x, w, preferred_element_type=jnp.float32)


def _dotg3(q, k):
    return jax.lax.dot_general(q, k, (((1,), (1,)), ((), ())),
                               preferred_element_type=jnp.float32)


def _mm_kernel(x_ref, w_ref, o_ref, acc_ref, *, nk):
    @pl.when(pl.program_id(2) == 0)
    def _init():
        acc_ref[...] = jnp.zeros_like(acc_ref)

    acc_ref[...] += _dot3(x_ref[...], w_ref[...])

    @pl.when(pl.program_id(2) == nk - 1)
    def _done():
        o_ref[...] = acc_ref[...].astype(o_ref.dtype)


def _mm(x, w, bm=1024, bn=2048, bk=512, out_dtype=None):
    m, k = x.shape
    _, n = w.shape
    bm, bn, bk = min(bm, m), min(bn, n), min(bk, k)
    nm, nn, nk = m // bm, n // bn, k // bk
    out_dtype = out_dtype or x.dtype
    return pl.pallas_call(
        functools.partial(_mm_kernel, nk=nk),
        grid=(nm, nn, nk),
        in_specs=[
            pl.BlockSpec((bm, bk), lambda i, j, kk: (i, kk)),
            pl.BlockSpec((bk, bn), lambda i, j, kk: (kk, j)),
        ],
        out_specs=pl.BlockSpec((bm, bn), lambda i, j, kk: (i, j)),
        out_shape=jax.ShapeDtypeStruct((m, n), out_dtype),
        scratch_shapes=[pltpu.VMEM((bm, bn), jnp.float32)],
        compiler_params=pltpu.CompilerParams(
            dimension_semantics=("parallel", "parallel", "arbitrary")),
    )(x, w)


def _glu_kernel(x_ref, wg_ref, wu_ref, h_ref):
    x = x_ref[...]
    g = _dot3(x, wg_ref[...])
    u = _dot3(x, wu_ref[...])
    h_ref[...] = (g * jax.nn.sigmoid(g) * u).astype(h_ref.dtype)


def _glu(x, wg, wu, bm=1024, bn=512):
    m, k = x.shape
    _, n = wg.shape
    bm, bn = min(bm, m), min(bn, n)
    return pl.pallas_call(
        _glu_kernel,
        grid=(m // bm, n // bn),
        in_specs=[
            pl.BlockSpec((bm, k), lambda i, j: (i, 0)),
            pl.BlockSpec((k, bn), lambda i, j: (0, j)),
            pl.BlockSpec((k, bn), lambda i, j: (0, j)),
        ],
        out_specs=pl.BlockSpec((bm, bn), lambda i, j: (i, j)),
        out_shape=jax.ShapeDtypeStruct((m, n), x.dtype),
        compiler_params=pltpu.CompilerParams(
            dimension_semantics=("parallel", "parallel")),
    )(x, wg, wu)


def _attn_blk_kernel(q_ref, k_ref, v_ref, o_ref, *, qi, bq, scale):
    s = _dotg3(q_ref[...], k_ref[...]) * scale
    pref = k_ref.shape[0]
    rows = qi * bq + jax.lax.broadcasted_iota(jnp.int32, (bq, pref), 0)
    cols = jax.lax.broadcasted_iota(jnp.int32, (bq, pref), 1)
    s = jnp.where(cols <= rows, s, -1e30)
    m = jnp.max(s, axis=1, keepdims=True)
    p = jnp.exp(s - m)
    a = p / jnp.sum(p, axis=1, keepdims=True)
    o_ref[...] = _dot3(a, v_ref[...])


def _attention(q, k, v, h, hd, bq=1024):
    t, d = q.shape
    bq = min(bq, t)
    scale = 1.0 / (hd ** 0.5)
    outs = []
    for qi in range(t // bq):
        pref = (qi + 1) * bq
        o = pl.pallas_call(
            functools.partial(_attn_blk_kernel, qi=qi, bq=bq, scale=scale),
            grid=(h,),
            in_specs=[
                pl.BlockSpec((bq, hd), lambda hh, _qi=qi: (_qi, hh)),
                pl.BlockSpec((pref, hd), lambda hh: (0, hh)),
                pl.BlockSpec((pref, hd), lambda hh: (0, hh)),
            ],
            out_specs=pl.BlockSpec((bq, hd), lambda hh: (0, hh)),
            out_shape=jax.ShapeDtypeStruct((bq, d), jnp.float32),
            compiler_params=pltpu.CompilerParams(
                dimension_semantics=("arbitrary",)),
        )(q, k, v)
        outs.append(o)
    return outs[0] if len(outs) == 1 else jnp.concatenate(outs, axis=0)


_SC_NC = 2
_SC_NS = 16
_SC_NW = _SC_NC * _SC_NS


def _sc_gather(table, idx):
    tt, d = table.shape
    kk = idx.shape[0]
    bpw = kk // _SC_NW
    mesh = plsc.VectorSubcoreMesh(core_axis_name="c", subcore_axis_name="s")

    @functools.partial(
        pl.kernel, mesh=mesh,
        out_type=jax.ShapeDtypeStruct((kk, d), jnp.float32),
        scratch_types=[
            pltpu.VMEM((bpw,), jnp.int32),
            pltpu.VMEM((bpw, d), jnp.float32),
            pltpu.SemaphoreType.DMA,
        ],
    )
    def gk(table_hbm, idx_hbm, out_hbm, idx_v, rows_v, sem):
        wid = lax.axis_index("s") * _SC_NC + lax.axis_index("c")
        base = wid * bpw
        pltpu.sync_copy(idx_hbm.at[pl.ds(base, bpw)], idx_v)
        pltpu.async_copy(table_hbm.at[idx_v], rows_v, sem).wait()
        pltpu.sync_copy(rows_v, out_hbm.at[pl.ds(base, bpw)])

    return gk(table, idx)


def _sc_scatter_combine(processed, cidx, idx, new):
    tt, d = processed.shape
    kk = idx.shape[0]
    bpw = kk // _SC_NW
    mesh = plsc.VectorSubcoreMesh(core_axis_name="c", subcore_axis_name="s")

    @functools.partial(
        pl.kernel, mesh=mesh,
        out_type=jax.ShapeDtypeStruct((tt, d), jnp.float32),
        scratch_types=[
            pltpu.VMEM((bpw,), jnp.int32),
            pltpu.VMEM((bpw,), jnp.int32),
            pltpu.VMEM((bpw, d), jnp.float32),
            pltpu.SemaphoreType.DMA,
        ],
    )
    def sk(proc_hbm, cidx_hbm, idx_hbm, new_hbm, out_hbm,
           cidx_v, idx_v, rows_v, sem):
        wid = lax.axis_index("s") * _SC_NC + lax.axis_index("c")
        base = wid * bpw
        pltpu.sync_copy(cidx_hbm.at[pl.ds(base, bpw)], cidx_v)
        pltpu.sync_copy(idx_hbm.at[pl.ds(base, bpw)], idx_v)
        pltpu.async_copy(proc_hbm.at[cidx_v], rows_v, sem).wait()
        pltpu.async_copy(rows_v, out_hbm.at[cidx_v], sem).wait()
        pltpu.sync_copy(new_hbm.at[pl.ds(base, bpw)], rows_v)
        pltpu.async_copy(rows_v, out_hbm.at[idx_v], sem).wait()

    return sk(processed, cidx, idx, new)


def _rms(x, w):
    return x * jax.lax.rsqrt(jnp.mean(x * x, axis=-1, keepdims=True) + EPS) * w


def _rope_cos_sin(t, hd):
    inv = 1.0 / (10000.0 ** (jnp.arange(0, hd, 2, dtype=jnp.float32) / hd))
    freqs = jnp.arange(t, dtype=jnp.float32)[:, None] * inv[None, :]
    cos = jnp.concatenate([jnp.cos(freqs), jnp.cos(freqs)], axis=-1)
    sin = jnp.concatenate([jnp.sin(freqs), jnp.sin(freqs)], axis=-1)
    return cos, sin


def _rope_tD(x, cos, sin, h, hd):
    t, d = x.shape
    xh = x.reshape(t, h, hd)
    x1 = xh[..., :hd // 2]
    x2 = xh[..., hd // 2:]
    rot = jnp.concatenate([-x2, x1], axis=-1)
    out = xh * cos[:, None, :] + rot * sin[:, None, :]
    return out.reshape(t, d)


def _decoder(x, p, pref, h, hd, cos, sin):
    xn = _rms(x, p[pref + 'ln1'])
    q = _mm(xn, p[pref + 'wq'])
    k = _mm(xn, p[pref + 'wk'])
    v = _mm(xn, p[pref + 'wv'])
    q = _rope_tD(q, cos, sin, h, hd)
    k = _rope_tD(k, cos, sin, h, hd)
    ao = _attention(q, k, v, h, hd)
    x = x + _mm(ao, p[pref + 'wo'])
    hn = _rms(x, p[pref + 'ln2'])
    hh = _glu(hn, p[pref + 'wg'], p[pref + 'wu'])
    return x + _mm(hh, p[pref + 'wd'])


def kernel(hidden_states, params):
    p = params
    b, t, d = hidden_states.shape
    x = hidden_states.reshape(t, d)
    h = 16
    hd = d // h
    cos, sin = _rope_cos_sin(t, hd)

    processed = _decoder(x, p, 'l1_', h, hd, cos, sin)
    pn = _rms(x, p['p_ln'])
    ph = _glu(pn, p['p_wg'], p['p_wu'])
    prior_out = _mm(ph, p['p_wd'])
    prior_hidden = x + prior_out

    prior_loss = jnp.mean((prior_hidden - processed) ** 2)

    actual = processed - x
    predicted = prior_hidden - x
    D_st = jnp.sum(actual ** 2, axis=-1) / d
    D_ch = jnp.sum((actual - predicted) ** 2, axis=-1) / d
    z_st = (D_st - jnp.mean(D_st)) / (jnp.std(D_st) + 1e-6)
    z_ch = (D_ch - jnp.mean(D_ch)) / (jnp.std(D_ch) + 1e-6)
    g_cont = jax.nn.sigmoid(BETA_CE * z_st - BETA_CU * z_ch)

    kk = max(1, int(t * 0.5))
    allscores, order = jax.lax.top_k(g_cont, t)
    gscores = allscores[:kk]
    topk_idx = order[:kk]
    comp_idx = order[kk:]

    binary = jnp.zeros((t,), jnp.float32).at[topk_idx].set(1.0)
    logits = x @ p['r_w']
    causal_loss = jnp.mean(jnp.maximum(logits, 0.0) - logits * binary
                           + jnp.log1p(jnp.exp(-jnp.abs(logits))))

    sel = _sc_gather(processed, topk_idx)
    out2 = _decoder(sel, p, 'l2_', h, hd, cos[:kk], sin[:kk])
    new = sel + gscores[:, None] * (out2 - sel)
    final = _sc_scatter_combine(processed, comp_idx, topk_idx, new)

    return final.reshape(b, t, d), prior_loss, causal_loss
